# baseline (speedup 1.0000x reference)
import functools

import jax
import jax.numpy as jnp
from jax import lax
from jax.experimental import pallas as pl
from jax.experimental.pallas import tpu as pltpu
from jax.experimental.pallas import tpu_sc as plsc

N = 10000
E = 320000
D = 128

NC = 2
NS = 16
NW = NC * NS

LPR = 128
ROWS_PER_TILE = 80
EDGES_PER_TILE = ROWS_PER_TILE * LPR
E_PAD = NW * EDGES_PER_TILE
N_PAD = 10240
ACC_ROWS = 10240

KB = 16


def _sc_mesh():
    return plsc.VectorSubcoreMesh(core_axis_name="c", subcore_axis_name="s")


def _sc_degree(dst2_hbm, out_hbm, deg_sh, idx_v, ones_v, zb_v):
    i32 = jnp.int32
    c = lax.axis_index("c")
    s = lax.axis_index("s")
    wid = c * i32(NS) + s

    def _z(i, _):
        zb_v[pl.ds(i * i32(16), 16)] = jnp.zeros((16,), jnp.float32)
        return _
    lax.fori_loop(i32(0), i32((N_PAD // NS) // 16), _z, i32(0))
    pltpu.sync_copy(zb_v, deg_sh.at[pl.ds(s * i32(N_PAD // NS), N_PAD // NS)])

    def _o(i, _):
        ones_v[pl.ds(i * i32(16), 16)] = jnp.ones((16,), jnp.float32)
        return _
    lax.fori_loop(i32(0), i32(LPR // 16), _o, i32(0))

    plsc.subcore_barrier()

    rbase = wid * i32(ROWS_PER_TILE)

    def _chunk(k, _):
        pltpu.sync_copy(dst2_hbm.at[pl.ds(rbase + k * i32(KB), KB)], idx_v)

        def _row(j, _):
            pltpu.sync_copy(ones_v, deg_sh.at[idx_v.at[j]], add=True)
            return _
        lax.fori_loop(i32(0), i32(KB), _row, i32(0))
        return _
    lax.fori_loop(i32(0), i32(ROWS_PER_TILE // KB), _chunk, i32(0))

    plsc.subcore_barrier()

    @pl.when(s == 0)
    def _():
        pltpu.sync_copy(deg_sh, out_hbm.at[c])


def _degree_partials(dst2):
    kern = pl.kernel(
        _sc_degree,
        out_type=jax.ShapeDtypeStruct((NC, N_PAD), jnp.float32),
        mesh=_sc_mesh(),
        name="sc_degree",
        scratch_types=[
            pltpu.VMEM_SHARED((N_PAD,), jnp.float32),
            pltpu.VMEM((KB, LPR), jnp.int32),
            pltpu.VMEM((LPR,), jnp.float32),
            pltpu.VMEM((N_PAD // NS,), jnp.float32),
        ],
    )
    return kern(dst2)


def _tc_transform(x_ref, w_ref, b_ref, degp_ref, g_ref, sl_ref):
    h = lax.dot_general(x_ref[...], w_ref[...], (((1,), (1,)), ((), ())),
                        preferred_element_type=jnp.float32)
    h = h + b_ref[...]
    deg = degp_ref[0] + degp_ref[1] + 1.0
    dinv = lax.rsqrt(deg)
    g = h * dinv
    g_ref[...] = g
    sl_ref[...] = g * dinv


def _transform(X, W, b2, degp3):
    R = 1000
    grid = (N // R,)
    return pl.pallas_call(
        _tc_transform,
        name="tc_transform",
        grid=grid,
        in_specs=[
            pl.BlockSpec((R, D), lambda i: (i, jnp.int32(0))),
            pl.BlockSpec((D, D), lambda i: (jnp.int32(0), jnp.int32(0))),
            pl.BlockSpec((1, D), lambda i: (jnp.int32(0), jnp.int32(0))),
            pl.BlockSpec((NC, R, 1), lambda i: (jnp.int32(0), i, jnp.int32(0))),
        ],
        out_specs=[
            pl.BlockSpec((R, D), lambda i: (i, jnp.int32(0))),
            pl.BlockSpec((R, D), lambda i: (i, jnp.int32(0))),
        ],
        out_shape=[
            jax.ShapeDtypeStruct((N, D), jnp.float32),
            jax.ShapeDtypeStruct((N, D), jnp.float32),
        ],
    )(X, W, b2, degp3)


def _sc_aggregate(g_hbm, src2_hbm, dst2_hbm, out_hbm,
                  acc_sh, src_a, dst_a, src_b, dst_b, rows0_v, rows1_v,
                  sem_g0, sem_g1, sem_s0, sem_s1, sem_i):
    i32 = jnp.int32
    c = lax.axis_index("c")
    s = lax.axis_index("s")
    wid = c * i32(NS) + s

    def _z(i, _):
        rows0_v[i // i32(D // 16), pl.ds((i % i32(D // 16)) * i32(16), 16)] = (
            jnp.zeros((16,), jnp.float32))
        return _
    lax.fori_loop(i32(0), i32(LPR * (D // 16)), _z, i32(0))
    zrows = ACC_ROWS // NS

    def _zc(k, _):
        pltpu.sync_copy(rows0_v,
                        acc_sh.at[pl.ds(s * i32(zrows) + k * i32(LPR), LPR)])
        return _
    lax.fori_loop(i32(0), i32(zrows // LPR), _zc, i32(0))

    plsc.subcore_barrier()

    rbase = wid * i32(ROWS_PER_TILE)
    nchunks = ROWS_PER_TILE // KB

    idx_bufs = [(src_a, dst_a), (src_b, dst_b)]
    pltpu.sync_copy(src2_hbm.at[pl.ds(rbase, KB)], src_a)
    pltpu.sync_copy(dst2_hbm.at[pl.ds(rbase, KB)], dst_a)
    pltpu.async_copy(g_hbm.at[src_a.at[i32(0)]], rows0_v, sem_g0)
    pltpu.async_copy(g_hbm.at[src_a.at[i32(1)]], rows1_v, sem_g1)

    for ci in range(nchunks):
        sv, dv = idx_bufs[ci % 2]
        nsv, ndv = idx_bufs[(ci + 1) % 2]
        if ci + 1 < nchunks:
            rb = rbase + i32((ci + 1) * KB)
            cpi_s = pltpu.async_copy(src2_hbm.at[pl.ds(rb, KB)], nsv, sem_i)
            cpi_d = pltpu.async_copy(dst2_hbm.at[pl.ds(rb, KB)], ndv, sem_i)

        def _pair(k, _, sv=sv, dv=dv):
            j0 = k * i32(2)
            pltpu.make_async_copy(g_hbm.at[sv.at[j0]], rows0_v,
                                  sem_g0).wait()
            s0 = pltpu.async_copy(rows0_v, acc_sh.at[dv.at[j0]], sem_s0,
                                  add=True)
            pltpu.make_async_copy(g_hbm.at[sv.at[j0 + i32(1)]], rows1_v,
                                  sem_g1).wait()
            s1 = pltpu.async_copy(rows1_v, acc_sh.at[dv.at[j0 + i32(1)]],
                                  sem_s1, add=True)
            s0.wait()
            pltpu.async_copy(g_hbm.at[sv.at[j0 + i32(2)]], rows0_v, sem_g0)
            s1.wait()
            pltpu.async_copy(g_hbm.at[sv.at[j0 + i32(3)]], rows1_v, sem_g1)
            return _
        lax.fori_loop(i32(0), i32(KB // 2 - 1), _pair, i32(0))

        jl = i32(KB - 2)
        pltpu.make_async_copy(g_hbm.at[sv.at[jl]], rows0_v, sem_g0).wait()
        s0 = pltpu.async_copy(rows0_v, acc_sh.at[dv.at[jl]], sem_s0,
                              add=True)
        pltpu.make_async_copy(g_hbm.at[sv.at[jl + i32(1)]], rows1_v,
                              sem_g1).wait()
        s1 = pltpu.async_copy(rows1_v, acc_sh.at[dv.at[jl + i32(1)]],
                              sem_s1, add=True)
        if ci + 1 < nchunks:
            cpi_s.wait()
            cpi_d.wait()
        s0.wait()
        s1.wait()
        if ci + 1 < nchunks:
            pltpu.async_copy(g_hbm.at[nsv.at[i32(0)]], rows0_v, sem_g0)
            pltpu.async_copy(g_hbm.at[nsv.at[i32(1)]], rows1_v, sem_g1)

    plsc.subcore_barrier()

    wrows = ACC_ROWS // NS
    pltpu.sync_copy(acc_sh.at[pl.ds(s * i32(wrows), wrows)],
                    out_hbm.at[c, pl.ds(s * i32(wrows), wrows)])


def _aggregate_partials(G, src2, dst2):
    kern = pl.kernel(
        _sc_aggregate,
        out_type=jax.ShapeDtypeStruct((NC, ACC_ROWS, D), jnp.float32),
        mesh=_sc_mesh(),
        name="sc_aggregate",
        scratch_types=[
            pltpu.VMEM_SHARED((ACC_ROWS, D), jnp.float32),
            pltpu.VMEM((KB, LPR), jnp.int32),
            pltpu.VMEM((KB, LPR), jnp.int32),
            pltpu.VMEM((KB, LPR), jnp.int32),
            pltpu.VMEM((KB, LPR), jnp.int32),
            pltpu.VMEM((LPR, D), jnp.float32),
            pltpu.VMEM((LPR, D), jnp.float32),
            pltpu.SemaphoreType.DMA,
            pltpu.SemaphoreType.DMA,
            pltpu.SemaphoreType.DMA,
            pltpu.SemaphoreType.DMA,
            pltpu.SemaphoreType.DMA,
        ],
    )
    return kern(G, src2, dst2)


def _tc_finalize(p_ref, sl_ref, degp_ref, o_ref):
    deg = degp_ref[0] + degp_ref[1] + 1.0
    dinv = lax.rsqrt(deg)
    acc = (p_ref[0] + p_ref[1]) * dinv + sl_ref[...]
    o_ref[...] = jnp.maximum(acc, 0.0)


def _finalize(P, SL, degp3):
    R = 1000
    grid = (N // R,)
    return pl.pallas_call(
        _tc_finalize,
        name="tc_finalize",
        grid=grid,
        in_specs=[
            pl.BlockSpec((NC, R, D), lambda i: (jnp.int32(0), i, jnp.int32(0))),
            pl.BlockSpec((R, D), lambda i: (i, jnp.int32(0))),
            pl.BlockSpec((NC, R, 1), lambda i: (jnp.int32(0), i, jnp.int32(0))),
        ],
        out_specs=pl.BlockSpec((R, D), lambda i: (i, jnp.int32(0))),
        out_shape=jax.ShapeDtypeStruct((N, D), jnp.float32),
    )(P, SL, degp3)


def kernel(X, edge_index, W, b):
    X = X.astype(jnp.float32)
    W = W.astype(jnp.float32)
    b2 = b.astype(jnp.float32).reshape(1, D)

    src = edge_index[0].astype(jnp.int32)
    dst = edge_index[1].astype(jnp.int32)
    pad = E_PAD - E
    iota = jnp.arange(pad, dtype=jnp.int32)
    src_p = jnp.concatenate([src, iota % N])
    dst_p = jnp.concatenate([dst, N + iota % (ACC_ROWS - N)])
    src2 = src_p.reshape(E_PAD // LPR, LPR)
    dst2 = dst_p.reshape(E_PAD // LPR, LPR)

    degp = _degree_partials(dst2)
    degp3 = degp.reshape(NC, N_PAD, 1)
    G, SL = _transform(X, W, b2, degp3)
    P = _aggregate_partials(G, src2, dst2)
    return _finalize(P, SL, degp3)


# speedup vs baseline: 1.0514x; 1.0514x over previous
import functools

import jax
import jax.numpy as jnp
from jax import lax
from jax.experimental import pallas as pl
from jax.experimental.pallas import tpu as pltpu
from jax.experimental.pallas import tpu_sc as plsc

N = 10000
E = 320000
D = 128

NC = 2
NS = 16
NW = NC * NS

LPR = 128
ROWS_PER_TILE = 80
EDGES_PER_TILE = ROWS_PER_TILE * LPR
E_PAD = NW * EDGES_PER_TILE
N_PAD = 10240
ACC_ROWS = 10240

KB = 16


def _sc_mesh():
    return plsc.VectorSubcoreMesh(core_axis_name="c", subcore_axis_name="s")


def _sc_degree(dst2_hbm, out_hbm, deg_sh, idx_v, ones_v, zb_v, sem_d):
    i32 = jnp.int32
    c = lax.axis_index("c")
    s = lax.axis_index("s")
    wid = c * i32(NS) + s

    def _z(i, _):
        zb_v[pl.ds(i * i32(16), 16)] = jnp.zeros((16,), jnp.float32)
        return _
    lax.fori_loop(i32(0), i32((N_PAD // NS) // 16), _z, i32(0))
    pltpu.sync_copy(zb_v, deg_sh.at[pl.ds(s * i32(N_PAD // NS), N_PAD // NS)])

    def _o(i, _):
        ones_v[pl.ds(i * i32(16), 16)] = jnp.ones((16,), jnp.float32)
        return _
    lax.fori_loop(i32(0), i32(LPR // 16), _o, i32(0))

    plsc.subcore_barrier()

    rbase = wid * i32(ROWS_PER_TILE)

    def _chunk(k, _):
        pltpu.sync_copy(dst2_hbm.at[pl.ds(rbase + k * i32(KB), KB)], idx_v)

        def _row(j, _):
            pltpu.async_copy(ones_v, deg_sh.at[idx_v.at[j]], sem_d, add=True)
            return _
        lax.fori_loop(i32(0), i32(KB), _row, i32(0))

        def _drain(j, _):
            pltpu.make_async_copy(ones_v, deg_sh.at[idx_v.at[j]],
                                  sem_d).wait()
            return _
        lax.fori_loop(i32(0), i32(KB), _drain, i32(0))
        return _
    lax.fori_loop(i32(0), i32(ROWS_PER_TILE // KB), _chunk, i32(0))

    plsc.subcore_barrier()

    @pl.when(s == 0)
    def _():
        pltpu.sync_copy(deg_sh, out_hbm.at[c])


def _degree_partials(dst2):
    kern = pl.kernel(
        _sc_degree,
        out_type=jax.ShapeDtypeStruct((NC, N_PAD), jnp.float32),
        mesh=_sc_mesh(),
        name="sc_degree",
        scratch_types=[
            pltpu.VMEM_SHARED((N_PAD,), jnp.float32),
            pltpu.VMEM((KB, LPR), jnp.int32),
            pltpu.VMEM((LPR,), jnp.float32),
            pltpu.VMEM((N_PAD // NS,), jnp.float32),
            pltpu.SemaphoreType.DMA,
        ],
    )
    return kern(dst2)


def _tc_transform(x_ref, w_ref, b_ref, degp_ref, g_ref, sl_ref):
    h = lax.dot_general(x_ref[...], w_ref[...], (((1,), (1,)), ((), ())),
                        preferred_element_type=jnp.float32)
    h = h + b_ref[...]
    deg = degp_ref[0] + degp_ref[1] + 1.0
    dinv = lax.rsqrt(deg)
    g = h * dinv
    g_ref[...] = g
    sl_ref[...] = g * dinv


def _transform(X, W, b2, degp3):
    R = 1000
    grid = (N // R,)
    return pl.pallas_call(
        _tc_transform,
        name="tc_transform",
        grid=grid,
        in_specs=[
            pl.BlockSpec((R, D), lambda i: (i, jnp.int32(0))),
            pl.BlockSpec((D, D), lambda i: (jnp.int32(0), jnp.int32(0))),
            pl.BlockSpec((1, D), lambda i: (jnp.int32(0), jnp.int32(0))),
            pl.BlockSpec((NC, R, 1), lambda i: (jnp.int32(0), i, jnp.int32(0))),
        ],
        out_specs=[
            pl.BlockSpec((R, D), lambda i: (i, jnp.int32(0))),
            pl.BlockSpec((R, D), lambda i: (i, jnp.int32(0))),
        ],
        out_shape=[
            jax.ShapeDtypeStruct((N, D), jnp.float32),
            jax.ShapeDtypeStruct((N, D), jnp.float32),
        ],
    )(X, W, b2, degp3)


def _sc_aggregate(g_hbm, src2_hbm, dst2_hbm, out_hbm,
                  acc_sh, src_v, dst_v, rows0_v, rows1_v, sem0, sem1):
    i32 = jnp.int32
    c = lax.axis_index("c")
    s = lax.axis_index("s")
    wid = c * i32(NS) + s

    def _z(i, _):
        rows0_v[i // i32(D // 16), pl.ds((i % i32(D // 16)) * i32(16), 16)] = (
            jnp.zeros((16,), jnp.float32))
        return _
    lax.fori_loop(i32(0), i32(LPR * (D // 16)), _z, i32(0))
    zrows = ACC_ROWS // NS

    def _zc(k, _):
        pltpu.sync_copy(rows0_v,
                        acc_sh.at[pl.ds(s * i32(zrows) + k * i32(LPR), LPR)])
        return _
    lax.fori_loop(i32(0), i32(zrows // LPR), _zc, i32(0))

    plsc.subcore_barrier()

    rbase = wid * i32(ROWS_PER_TILE)

    def _chunk(cidx, _):
        rb = rbase + cidx * i32(KB)
        pltpu.sync_copy(src2_hbm.at[pl.ds(rb, KB)], src_v)
        pltpu.sync_copy(dst2_hbm.at[pl.ds(rb, KB)], dst_v)
        pltpu.async_copy(g_hbm.at[src_v.at[i32(0)]], rows0_v, sem0)

        def _pair(k, _):
            j0 = k * i32(2)
            pltpu.make_async_copy(g_hbm.at[src_v.at[j0]], rows0_v,
                                  sem0).wait()
            pltpu.async_copy(g_hbm.at[src_v.at[j0 + i32(1)]], rows1_v, sem1)
            pltpu.sync_copy(rows0_v, acc_sh.at[dst_v.at[j0]], add=True)
            pltpu.make_async_copy(g_hbm.at[src_v.at[j0 + i32(1)]],
                                  rows1_v, sem1).wait()

            @pl.when(k < i32(KB // 2 - 1))
            def _prefetch():
                pltpu.async_copy(g_hbm.at[src_v.at[j0 + i32(2)]], rows0_v,
                                 sem0)
            pltpu.sync_copy(rows1_v, acc_sh.at[dst_v.at[j0 + i32(1)]],
                            add=True)
            return _
        lax.fori_loop(i32(0), i32(KB // 2), _pair, i32(0))
        return _
    lax.fori_loop(i32(0), i32(ROWS_PER_TILE // KB), _chunk, i32(0))

    plsc.subcore_barrier()

    wrows = ACC_ROWS // NS
    pltpu.sync_copy(acc_sh.at[pl.ds(s * i32(wrows), wrows)],
                    out_hbm.at[c, pl.ds(s * i32(wrows), wrows)])


def _aggregate_partials(G, src2, dst2):
    kern = pl.kernel(
        _sc_aggregate,
        out_type=jax.ShapeDtypeStruct((NC, ACC_ROWS, D), jnp.float32),
        mesh=_sc_mesh(),
        name="sc_aggregate",
        scratch_types=[
            pltpu.VMEM_SHARED((ACC_ROWS, D), jnp.float32),
            pltpu.VMEM((KB, LPR), jnp.int32),
            pltpu.VMEM((KB, LPR), jnp.int32),
            pltpu.VMEM((LPR, D), jnp.float32),
            pltpu.VMEM((LPR, D), jnp.float32),
            pltpu.SemaphoreType.DMA,
            pltpu.SemaphoreType.DMA,
        ],
    )
    return kern(G, src2, dst2)


def _tc_finalize(p_ref, sl_ref, degp_ref, o_ref):
    deg = degp_ref[0] + degp_ref[1] + 1.0
    dinv = lax.rsqrt(deg)
    acc = (p_ref[0] + p_ref[1]) * dinv + sl_ref[...]
    o_ref[...] = jnp.maximum(acc, 0.0)


def _finalize(P, SL, degp3):
    R = 1000
    grid = (N // R,)
    return pl.pallas_call(
        _tc_finalize,
        name="tc_finalize",
        grid=grid,
        in_specs=[
            pl.BlockSpec((NC, R, D), lambda i: (jnp.int32(0), i, jnp.int32(0))),
            pl.BlockSpec((R, D), lambda i: (i, jnp.int32(0))),
            pl.BlockSpec((NC, R, 1), lambda i: (jnp.int32(0), i, jnp.int32(0))),
        ],
        out_specs=pl.BlockSpec((R, D), lambda i: (i, jnp.int32(0))),
        out_shape=jax.ShapeDtypeStruct((N, D), jnp.float32),
    )(P, SL, degp3)


def kernel(X, edge_index, W, b):
    X = X.astype(jnp.float32)
    W = W.astype(jnp.float32)
    b2 = b.astype(jnp.float32).reshape(1, D)

    src = edge_index[0].astype(jnp.int32)
    dst = edge_index[1].astype(jnp.int32)
    pad = E_PAD - E
    iota = jnp.arange(pad, dtype=jnp.int32)
    src_p = jnp.concatenate([src, iota % N])
    dst_p = jnp.concatenate([dst, N + iota % (ACC_ROWS - N)])
    src2 = src_p.reshape(E_PAD // LPR, LPR)
    dst2 = dst_p.reshape(E_PAD // LPR, LPR)

    degp = _degree_partials(dst2)
    degp3 = degp.reshape(NC, N_PAD, 1)
    G, SL = _transform(X, W, b2, degp3)
    P = _aggregate_partials(G, src2, dst2)
    return _finalize(P, SL, degp3)


# speedup vs baseline: 1.0544x; 1.0029x over previous
import functools

import jax
import jax.numpy as jnp
from jax import lax
from jax.experimental import pallas as pl
from jax.experimental.pallas import tpu as pltpu
from jax.experimental.pallas import tpu_sc as plsc

N = 10000
E = 320000
D = 128

NC = 2
NS = 16
NW = NC * NS

LPR = 128
ROWS_PER_TILE = 80
EDGES_PER_TILE = ROWS_PER_TILE * LPR
E_PAD = NW * EDGES_PER_TILE
N_PAD = 10240
ACC_ROWS = 10240

KB = 16


def _sc_mesh():
    return plsc.VectorSubcoreMesh(core_axis_name="c", subcore_axis_name="s")


def _sc_degree(dst2_hbm, out_hbm, deg_sh, idx_v, ones_v, zb_v, sem_d):
    i32 = jnp.int32
    c = lax.axis_index("c")
    s = lax.axis_index("s")
    wid = c * i32(NS) + s

    def _z(i, _):
        zb_v[pl.ds(i * i32(16), 16)] = jnp.zeros((16,), jnp.float32)
        return _
    lax.fori_loop(i32(0), i32((N_PAD // NS) // 16), _z, i32(0))
    pltpu.sync_copy(zb_v, deg_sh.at[pl.ds(s * i32(N_PAD // NS), N_PAD // NS)])

    def _o(i, _):
        ones_v[pl.ds(i * i32(16), 16)] = jnp.ones((16,), jnp.float32)
        return _
    lax.fori_loop(i32(0), i32(LPR // 16), _o, i32(0))

    plsc.subcore_barrier()

    rbase = wid * i32(ROWS_PER_TILE)

    def _chunk(k, _):
        pltpu.sync_copy(dst2_hbm.at[pl.ds(rbase + k * i32(KB), KB)], idx_v)

        def _row(j, _):
            pltpu.async_copy(ones_v, deg_sh.at[idx_v.at[j]], sem_d, add=True)
            return _
        lax.fori_loop(i32(0), i32(KB), _row, i32(0))

        def _drain(j, _):
            pltpu.make_async_copy(ones_v, deg_sh.at[idx_v.at[j]],
                                  sem_d).wait()
            return _
        lax.fori_loop(i32(0), i32(KB), _drain, i32(0))
        return _
    lax.fori_loop(i32(0), i32(ROWS_PER_TILE // KB), _chunk, i32(0))

    plsc.subcore_barrier()

    @pl.when(s == 0)
    def _():
        pltpu.sync_copy(deg_sh, out_hbm.at[c])


def _degree_partials(dst2):
    kern = pl.kernel(
        _sc_degree,
        out_type=jax.ShapeDtypeStruct((NC, N_PAD), jnp.float32),
        mesh=_sc_mesh(),
        name="sc_degree",
        scratch_types=[
            pltpu.VMEM_SHARED((N_PAD,), jnp.float32),
            pltpu.VMEM((KB, LPR), jnp.int32),
            pltpu.VMEM((LPR,), jnp.float32),
            pltpu.VMEM((N_PAD // NS,), jnp.float32),
            pltpu.SemaphoreType.DMA,
        ],
    )
    return kern(dst2)


def _tc_transform(x_ref, w_ref, b_ref, degp_ref, g_ref):
    h = lax.dot_general(x_ref[...], w_ref[...], (((1,), (1,)), ((), ())),
                        preferred_element_type=jnp.float32)
    h = h + b_ref[...]
    deg = degp_ref[0] + degp_ref[1] + 1.0
    dinv = lax.rsqrt(deg)
    g_ref[...] = h * dinv


def _transform(X, W, b2, degp3):
    R = 1000
    grid = (N // R,)
    return pl.pallas_call(
        _tc_transform,
        name="tc_transform",
        grid=grid,
        in_specs=[
            pl.BlockSpec((R, D), lambda i: (i, jnp.int32(0))),
            pl.BlockSpec((D, D), lambda i: (jnp.int32(0), jnp.int32(0))),
            pl.BlockSpec((1, D), lambda i: (jnp.int32(0), jnp.int32(0))),
            pl.BlockSpec((NC, R, 1), lambda i: (jnp.int32(0), i, jnp.int32(0))),
        ],
        out_specs=pl.BlockSpec((R, D), lambda i: (i, jnp.int32(0))),
        out_shape=jax.ShapeDtypeStruct((N, D), jnp.float32),
    )(X, W, b2, degp3)


def _sc_aggregate(g_hbm, src2_hbm, dst2_hbm, out_hbm,
                  acc_sh, src_v, dst_v, rows0_v, rows1_v, sem0, sem1):
    i32 = jnp.int32
    c = lax.axis_index("c")
    s = lax.axis_index("s")
    wid = c * i32(NS) + s

    def _z(i, _):
        rows0_v[i // i32(D // 16), pl.ds((i % i32(D // 16)) * i32(16), 16)] = (
            jnp.zeros((16,), jnp.float32))
        return _
    lax.fori_loop(i32(0), i32(LPR * (D // 16)), _z, i32(0))
    zrows = ACC_ROWS // NS

    def _zc(k, _):
        pltpu.sync_copy(rows0_v,
                        acc_sh.at[pl.ds(s * i32(zrows) + k * i32(LPR), LPR)])
        return _
    lax.fori_loop(i32(0), i32(zrows // LPR), _zc, i32(0))

    plsc.subcore_barrier()

    rbase = wid * i32(ROWS_PER_TILE)

    def _chunk(cidx, _):
        rb = rbase + cidx * i32(KB)
        pltpu.sync_copy(src2_hbm.at[pl.ds(rb, KB)], src_v)
        pltpu.sync_copy(dst2_hbm.at[pl.ds(rb, KB)], dst_v)
        pltpu.async_copy(g_hbm.at[src_v.at[i32(0)]], rows0_v, sem0)

        def _pair(k, _):
            j0 = k * i32(2)
            pltpu.make_async_copy(g_hbm.at[src_v.at[j0]], rows0_v,
                                  sem0).wait()
            pltpu.async_copy(g_hbm.at[src_v.at[j0 + i32(1)]], rows1_v, sem1)
            pltpu.sync_copy(rows0_v, acc_sh.at[dst_v.at[j0]], add=True)
            pltpu.make_async_copy(g_hbm.at[src_v.at[j0 + i32(1)]],
                                  rows1_v, sem1).wait()

            @pl.when(k < i32(KB // 2 - 1))
            def _prefetch():
                pltpu.async_copy(g_hbm.at[src_v.at[j0 + i32(2)]], rows0_v,
                                 sem0)
            pltpu.sync_copy(rows1_v, acc_sh.at[dst_v.at[j0 + i32(1)]],
                            add=True)
            return _
        lax.fori_loop(i32(0), i32(KB // 2), _pair, i32(0))
        return _
    lax.fori_loop(i32(0), i32(ROWS_PER_TILE // KB), _chunk, i32(0))

    plsc.subcore_barrier()

    wrows = ACC_ROWS // NS
    pltpu.sync_copy(acc_sh.at[pl.ds(s * i32(wrows), wrows)],
                    out_hbm.at[c, pl.ds(s * i32(wrows), wrows)])


def _aggregate_partials(G, src2, dst2):
    kern = pl.kernel(
        _sc_aggregate,
        out_type=jax.ShapeDtypeStruct((NC, ACC_ROWS, D), jnp.float32),
        mesh=_sc_mesh(),
        name="sc_aggregate",
        scratch_types=[
            pltpu.VMEM_SHARED((ACC_ROWS, D), jnp.float32),
            pltpu.VMEM((KB, LPR), jnp.int32),
            pltpu.VMEM((KB, LPR), jnp.int32),
            pltpu.VMEM((LPR, D), jnp.float32),
            pltpu.VMEM((LPR, D), jnp.float32),
            pltpu.SemaphoreType.DMA,
            pltpu.SemaphoreType.DMA,
        ],
    )
    return kern(G, src2, dst2)


def _tc_finalize(p_ref, g_ref, degp_ref, o_ref):
    deg = degp_ref[0] + degp_ref[1] + 1.0
    dinv = lax.rsqrt(deg)
    acc = (p_ref[0] + p_ref[1] + g_ref[...]) * dinv
    o_ref[...] = jnp.maximum(acc, 0.0)


def _finalize(P, SL, degp3):
    R = 1000
    grid = (N // R,)
    return pl.pallas_call(
        _tc_finalize,
        name="tc_finalize",
        grid=grid,
        in_specs=[
            pl.BlockSpec((NC, R, D), lambda i: (jnp.int32(0), i, jnp.int32(0))),
            pl.BlockSpec((R, D), lambda i: (i, jnp.int32(0))),
            pl.BlockSpec((NC, R, 1), lambda i: (jnp.int32(0), i, jnp.int32(0))),
        ],
        out_specs=pl.BlockSpec((R, D), lambda i: (i, jnp.int32(0))),
        out_shape=jax.ShapeDtypeStruct((N, D), jnp.float32),
    )(P, SL, degp3)


def kernel(X, edge_index, W, b):
    X = X.astype(jnp.float32)
    W = W.astype(jnp.float32)
    b2 = b.astype(jnp.float32).reshape(1, D)

    src = edge_index[0].astype(jnp.int32)
    dst = edge_index[1].astype(jnp.int32)
    pad = E_PAD - E
    iota = jnp.arange(pad, dtype=jnp.int32)
    src_p = jnp.concatenate([src, iota % N])
    dst_p = jnp.concatenate([dst, N + iota % (ACC_ROWS - N)])
    src2 = src_p.reshape(E_PAD // LPR, LPR)
    dst2 = dst_p.reshape(E_PAD // LPR, LPR)

    degp = _degree_partials(dst2)
    degp3 = degp.reshape(NC, N_PAD, 1)
    G = _transform(X, W, b2, degp3)
    P = _aggregate_partials(G, src2, dst2)
    return _finalize(P, G, degp3)
